# baseline (speedup 1.0000x reference)
import functools

import jax
import jax.numpy as jnp
from jax import lax
from jax.experimental import pallas as pl
from jax.experimental.pallas import tpu as pltpu
from jax.experimental.pallas import tpu_sc as plsc

_B, _S, _D = 4, 4096, 1024
_NC, _NS = 2, 16
_NW = _NC * _NS
_SPW = _S // _NW
_CH = 16
_NCHUNK = _SPW // _CH
_GRP = _D // 16
_NBUF = 4

_mesh = plsc.VectorSubcoreMesh(core_axis_name="c", subcore_axis_name="s")


@functools.partial(
    pl.kernel,
    out_type=jax.ShapeDtypeStruct((_B, _S, _D), jnp.float32),
    mesh=_mesh,
    compiler_params=pltpu.CompilerParams(use_tc_tiling_on_sc=True),
    scratch_types=(
        [pltpu.VMEM((_CH, _D), jnp.float32) for _ in range(_NBUF)]
        + [pltpu.VMEM((_CH, _D), jnp.float32) for _ in range(2)]
        + [pltpu.SemaphoreType.DMA for _ in range(2 * _NBUF + 2)]
    ),
)
def _sc_add(x_hbm, emb_hbm, out_hbm, *bufs):
    xbuf = bufs[:_NBUF]
    ebuf = bufs[_NBUF:_NBUF + 2]
    isem = bufs[_NBUF + 2:2 * _NBUF + 2]
    osem = bufs[2 * _NBUF + 2:3 * _NBUF + 2]
    esem = bufs[3 * _NBUF + 2:3 * _NBUF + 4]
    wid = lax.axis_index("s") * _NC + lax.axis_index("c")
    base = wid * _SPW
    in_d = [None] * _NBUF
    out_d = [None] * _NBUF
    emb_d = [None, None]

    def xsl(t):
        ci, b = divmod(t, _B)
        return x_hbm.at[b, pl.ds(base + ci * _CH, _CH)]

    def osl(t):
        ci, b = divmod(t, _B)
        return out_hbm.at[b, pl.ds(base + ci * _CH, _CH)]

    _H = _CH // 2

    def issue_in(t, p):
        ci, b = divmod(t, _B)
        s0 = base + ci * _CH
        return (
            pltpu.async_copy(x_hbm.at[b, pl.ds(s0, _H)],
                             xbuf[p].at[pl.ds(0, _H)], isem[p]),
            pltpu.async_copy(x_hbm.at[b, pl.ds(s0 + _H, _H)],
                             xbuf[p].at[pl.ds(_H, _H)], isem[p]),
        )

    ntiles = _NCHUNK * _B
    emb_d[0] = pltpu.async_copy(emb_hbm.at[pl.ds(base, _CH)], ebuf[0], esem[0])
    for t0 in range(_NBUF - 1):
        in_d[t0] = issue_in(t0, t0)

    for t in range(ntiles):
        p = t % _NBUF
        ci, b = divmod(t, _B)
        q = ci & 1
        if b == 0:
            if ci + 1 < _NCHUNK:
                emb_d[1 - q] = pltpu.async_copy(
                    emb_hbm.at[pl.ds(base + (ci + 1) * _CH, _CH)],
                    ebuf[1 - q], esem[1 - q])
            emb_d[q].wait()
        in_d[p][0].wait()
        in_d[p][1].wait()

        xb, eb = xbuf[p], ebuf[q]

        @plsc.parallel_loop(0, _CH * _GRP, step=1, unroll=16)
        def _add(i):
            r = i >> 6
            c = (i & (_GRP - 1)) * 16
            xb[r, pl.ds(c, 16)] = xb[r, pl.ds(c, 16)] + eb[r, pl.ds(c, 16)]

        out_d[p] = pltpu.async_copy(xbuf[p], osl(t), osem[p])
        nxt = t + _NBUF - 1
        if nxt < ntiles:
            np_ = nxt % _NBUF
            if out_d[np_] is not None:
                out_d[np_].wait()
            in_d[np_] = issue_in(nxt, np_)

    for k in range(max(0, ntiles - _NBUF), ntiles):
        out_d[k % _NBUF].wait()


@jax.jit
def kernel(x, emb):
    return _sc_add(x, emb)


# speedup vs baseline: 1.0507x; 1.0507x over previous
import functools

import jax
import jax.numpy as jnp
from jax import lax
from jax.experimental import pallas as pl
from jax.experimental.pallas import tpu as pltpu
from jax.experimental.pallas import tpu_sc as plsc

_B, _S, _D = 4, 4096, 1024
_NC, _NS = 2, 16
_NW = _NC * _NS
_SPW = _S // _NW
_CH = 8
_NCHUNK = _SPW // _CH
_GRP = _D // 16
_NSET = 3

_mesh = plsc.VectorSubcoreMesh(core_axis_name="c", subcore_axis_name="s")


@functools.partial(
    pl.kernel,
    out_type=jax.ShapeDtypeStruct((_B, _S, _D), jnp.float32),
    mesh=_mesh,
    compiler_params=pltpu.CompilerParams(use_tc_tiling_on_sc=True),
    scratch_types=(
        [pltpu.VMEM((_CH, _D), jnp.float32) for _ in range(_NSET * _B)]
        + [pltpu.VMEM((_CH, _D), jnp.float32) for _ in range(2)]
        + [pltpu.SemaphoreType.DMA for _ in range(2 * _NSET * _B + 2)]
    ),
)
def _sc_add(x_hbm, emb_hbm, out_hbm, *bufs):
    nxb = _NSET * _B
    xbuf = [bufs[s * _B:(s + 1) * _B] for s in range(_NSET)]
    ebuf = bufs[nxb:nxb + 2]
    isem = [bufs[nxb + 2 + s * _B:nxb + 2 + (s + 1) * _B] for s in range(_NSET)]
    osem = [bufs[nxb + 2 + nxb + s * _B:nxb + 2 + nxb + (s + 1) * _B]
            for s in range(_NSET)]
    esem = bufs[2 * nxb + 2:2 * nxb + 4]
    wid = lax.axis_index("s") * _NC + lax.axis_index("c")
    base = wid * _SPW
    in_d = [[None] * _B for _ in range(_NSET)]
    out_d = [[None] * _B for _ in range(_NSET)]
    emb_d = [None, None]

    def xsl(ci, b):
        return x_hbm.at[b, pl.ds(base + ci * _CH, _CH)]

    def osl(ci, b):
        return out_hbm.at[b, pl.ds(base + ci * _CH, _CH)]

    emb_d[0] = pltpu.async_copy(emb_hbm.at[pl.ds(base, _CH)], ebuf[0], esem[0])
    for ci0 in range(2):
        for b in range(_B):
            in_d[ci0][b] = pltpu.async_copy(xsl(ci0, b), xbuf[ci0][b],
                                            isem[ci0][b])

    for ci in range(_NCHUNK):
        P = ci % _NSET
        Q = ci & 1
        if ci + 1 < _NCHUNK:
            emb_d[1 - Q] = pltpu.async_copy(
                emb_hbm.at[pl.ds(base + (ci + 1) * _CH, _CH)],
                ebuf[1 - Q], esem[1 - Q])
        emb_d[Q].wait()
        if ci + 2 < _NCHUNK:
            S = (ci + 2) % _NSET
            for b in range(_B):
                if out_d[S][b] is not None:
                    out_d[S][b].wait()
                in_d[S][b] = pltpu.async_copy(xsl(ci + 2, b), xbuf[S][b],
                                              isem[S][b])
        for b in range(_B):
            in_d[P][b].wait()

        x0, x1, x2, x3 = xbuf[P]
        eb = ebuf[Q]

        @plsc.parallel_loop(0, _CH * _GRP, step=1, unroll=8)
        def _add(i):
            r = i >> 6
            c = (i & (_GRP - 1)) * 16
            ev = eb[r, pl.ds(c, 16)]
            x0[r, pl.ds(c, 16)] = x0[r, pl.ds(c, 16)] + ev
            x1[r, pl.ds(c, 16)] = x1[r, pl.ds(c, 16)] + ev
            x2[r, pl.ds(c, 16)] = x2[r, pl.ds(c, 16)] + ev
            x3[r, pl.ds(c, 16)] = x3[r, pl.ds(c, 16)] + ev

        for b in range(_B):
            out_d[P][b] = pltpu.async_copy(xbuf[P][b], osl(ci, b), osem[P][b])

    for ci in range(_NCHUNK - _NSET, _NCHUNK):
        for b in range(_B):
            out_d[ci % _NSET][b].wait()


@jax.jit
def kernel(x, emb):
    return _sc_add(x, emb)
